# Initial kernel scaffold; baseline (speedup 1.0000x reference)
import jax
import jax.numpy as jnp
from jax.experimental import pallas as pl


def kernel(x, edge_index, W1, b1, W2, b2):
    raise NotImplementedError("write your pallas kernel here")


# speedup vs baseline: 5.1884x; 5.1884x over previous
import functools

import jax
import jax.numpy as jnp
from jax import lax
from jax.experimental import pallas as pl
from jax.experimental.pallas import tpu as pltpu
from jax.experimental.pallas import tpu_sc as plsc

_NC = 2
_NS = 16
_CH = 128
_NQ = 2
_DH = 128


@functools.lru_cache(maxsize=None)
def _make_deg_kernel(E, NPAD):
  EC = E // _NC
  nch = EC // _CH
  per = -(-nch // _NS)
  NZ = NPAD // _NS
  mesh = plsc.VectorSubcoreMesh(core_axis_name="c", subcore_axis_name="s")

  @functools.partial(
      pl.kernel, mesh=mesh,
      out_type=jax.ShapeDtypeStruct((_NC * NPAD,), jnp.float32),
      scratch_types=[
          pltpu.VMEM((_CH,), jnp.int32),
          pltpu.VMEM((_CH,), jnp.float32),
          pltpu.VMEM((NZ,), jnp.float32),
          pltpu.VMEM_SHARED((NPAD,), jnp.float32),
      ],
  )
  def deg_k(dst_hbm, zeros_hbm, out_hbm, idxv, onesv, zbuf, acc):
    c = lax.axis_index("c")
    s = lax.axis_index("s")
    pltpu.sync_copy(zeros_hbm, zbuf)
    pltpu.sync_copy(zbuf, acc.at[pl.ds(s * NZ, NZ)])
    for i in range(_CH // 16):
      onesv[pl.ds(i * 16, 16)] = jnp.full((16,), 1.0, jnp.float32)
    plsc.subcore_barrier()

    def body(k, carry):
      ch = s + k * _NS

      @pl.when(ch < nch)
      def _():
        base = c * EC + ch * _CH
        pltpu.sync_copy(dst_hbm.at[pl.ds(base, _CH)], idxv)
        pltpu.sync_copy(onesv, acc.at[idxv], add=True)

      return carry

    lax.fori_loop(0, per, body, 0)
    plsc.subcore_barrier()
    pltpu.sync_copy(acc.at[pl.ds(s * NZ, NZ)], zbuf)
    pltpu.sync_copy(zbuf, out_hbm.at[pl.ds(c * NPAD + s * NZ, NZ)])

  return deg_k


@functools.lru_cache(maxsize=None)
def _make_scatter_kernel(E, N):
  nch = E // _CH
  per = -(-nch // _NS)
  HALF = N // _NC
  ACC = HALF + 8
  NRA = (HALF // _NS + 7) // 8 * 8
  NRL = HALF - (_NS - 1) * NRA
  NZA = (ACC // _NS + 7) // 8 * 8
  NZL = ACC - (_NS - 1) * NZA
  mesh = plsc.VectorSubcoreMesh(core_axis_name="c", subcore_axis_name="s")

  @functools.partial(
      pl.kernel, mesh=mesh,
      out_type=jax.ShapeDtypeStruct((_NQ * N, _DH), jnp.float32),
      scratch_types=[
          pltpu.VMEM((_CH,), jnp.int32),
          pltpu.VMEM((_CH,), jnp.int32),
          pltpu.VMEM((_CH,), jnp.int32),
          pltpu.VMEM((_CH, _DH), jnp.float32),
          pltpu.VMEM((NZA, _DH), jnp.float32),
          pltpu.VMEM_SHARED((ACC, _DH), jnp.float32),
          pltpu.SemaphoreType.DMA,
      ],
  )
  def scat_k(g_hbm, src_hbm, dst_hbm, zeros_hbm, out_hbm,
             srcv, gidxv, dstv, rows, zbuf, acc, sem):
    c = lax.axis_index("c")
    s = lax.axis_index("s")
    last = s == _NS - 1
    lo = c * HALF
    pltpu.sync_copy(zeros_hbm, zbuf)

    for p in range(_NQ):
      @pl.when(~last)
      def _():
        pltpu.sync_copy(zbuf, acc.at[pl.ds(s * NZA, NZA)])

      @pl.when(last)
      def _():
        pltpu.sync_copy(zbuf.at[pl.ds(0, NZL)], acc.at[pl.ds(s * NZA, NZL)])

      plsc.subcore_barrier()

      def body(k, carry):
        ch = s + k * _NS

        @pl.when(ch < nch)
        def _():
          base = ch * _CH
          pltpu.sync_copy(src_hbm.at[pl.ds(base, _CH)], srcv)
          pltpu.sync_copy(dst_hbm.at[pl.ds(base, _CH)], dstv)
          for i in range(_CH // 16):
            sl = pl.ds(i * 16, 16)
            gidxv[sl] = srcv[sl] * _NQ + p
            dl = dstv[sl] - lo
            ok = (dl >= 0) & (dl < HALF)
            dstv[sl] = jnp.where(ok, dl, HALF)
          pltpu.async_copy(g_hbm.at[gidxv], rows, sem).wait()
          pltpu.sync_copy(rows, acc.at[dstv], add=True)

        return carry

      lax.fori_loop(0, per, body, 0)
      plsc.subcore_barrier()

      obase = pl.multiple_of(p * N + c * HALF + s * NRA, 8)

      @pl.when(~last)
      def _():
        pltpu.sync_copy(acc.at[pl.ds(s * NRA, NRA)], zbuf.at[pl.ds(0, NRA)])
        pltpu.sync_copy(zbuf.at[pl.ds(0, NRA)], out_hbm.at[pl.ds(obase, NRA)])
        pltpu.sync_copy(zeros_hbm.at[pl.ds(0, NRA)], zbuf.at[pl.ds(0, NRA)])

      @pl.when(last)
      def _():
        pltpu.sync_copy(acc.at[pl.ds(s * NRA, NRL)], zbuf.at[pl.ds(0, NRL)])
        pltpu.sync_copy(zbuf.at[pl.ds(0, NRL)], out_hbm.at[pl.ds(obase, NRL)])
        pltpu.sync_copy(zeros_hbm.at[pl.ds(0, NRL)], zbuf.at[pl.ds(0, NRL)])

  return scat_k


def _tc_first(x_ref, w_ref, d0_ref, d1_ref, g_ref, dis_ref):
  dis = lax.rsqrt(d0_ref[...] + d1_ref[...] + 1.0)
  z = jnp.dot(x_ref[...], w_ref[...], preferred_element_type=jnp.float32)
  g_ref[...] = z * dis
  dis_ref[...] = dis


def _tc_mid(s0_ref, s1_ref, g_ref, dis_ref, w_ref, b_ref, h_ref, gn_ref):
  dis = dis_ref[...]
  h = dis * (jnp.concatenate([s0_ref[...], s1_ref[...]], axis=1)
             + g_ref[...]) + b_ref[...]
  h_ref[...] = h
  z = jnp.dot(h, w_ref[...], preferred_element_type=jnp.float32)
  gn_ref[...] = z * dis


def _tc_stats(h_ref, sum_ref, sq_ref):
  i = pl.program_id(0)
  h = h_ref[...]
  ps = jnp.sum(h, axis=0, keepdims=True)
  pq = jnp.sum(h * h, axis=0, keepdims=True)

  @pl.when(i == 0)
  def _():
    sum_ref[...] = ps
    sq_ref[...] = pq

  @pl.when(i > 0)
  def _():
    sum_ref[...] += ps
    sq_ref[...] += pq


def _make_tc_final(N):
  def _tc_final(h_ref, sum_ref, sq_ref, out_ref):
    n = float(N)
    mean = sum_ref[...] / n
    var = (sq_ref[...] - sum_ref[...] * mean) / (n - 1.0)
    out_ref[...] = (h_ref[...] - mean) * lax.rsqrt(var)

  return _tc_final


def kernel(x, edge_index, W1, b1, W2, b2):
  N, D = x.shape
  E = edge_index.shape[1]
  RB = 1000
  nb = N // RB
  NPAD = ((N + 16 * 8 - 1) // (16 * 8)) * (16 * 8)
  ACC = N // _NC + 8
  NZA = (ACC // _NS + 7) // 8 * 8

  src = edge_index[0]
  dst = edge_index[1]
  zeros_deg = jnp.zeros((NPAD // _NS,), jnp.float32)
  zeros_acc = jnp.zeros((NZA, _DH), jnp.float32)

  degp = _make_deg_kernel(E, NPAD)(dst, zeros_deg)
  d0 = degp[:N].reshape(N, 1)
  d1 = degp[NPAD:NPAD + N].reshape(N, 1)

  row = lambda i: (i, 0)
  full = lambda i: (0, 0)
  g1, dis = pl.pallas_call(
      _tc_first,
      grid=(nb,),
      in_specs=[
          pl.BlockSpec((RB, D), row),
          pl.BlockSpec((D, D), full),
          pl.BlockSpec((RB, 1), row),
          pl.BlockSpec((RB, 1), row),
      ],
      out_specs=[
          pl.BlockSpec((RB, D), row),
          pl.BlockSpec((RB, 1), row),
      ],
      out_shape=[
          jax.ShapeDtypeStruct((N, D), jnp.float32),
          jax.ShapeDtypeStruct((N, 1), jnp.float32),
      ],
  )(x, W1, d0, d1)

  scat = _make_scatter_kernel(E, N)

  quad = [lambda i, q=q: (q * nb + i, 0) for q in range(_NQ)]
  mid = pl.pallas_call(
      _tc_mid,
      grid=(nb,),
      in_specs=[pl.BlockSpec((RB, _DH), quad[q]) for q in range(_NQ)] + [
          pl.BlockSpec((RB, D), row),
          pl.BlockSpec((RB, 1), row),
          pl.BlockSpec((D, D), full),
          pl.BlockSpec((1, D), full),
      ],
      out_specs=[
          pl.BlockSpec((RB, D), row),
          pl.BlockSpec((RB, D), row),
      ],
      out_shape=[
          jax.ShapeDtypeStruct((N, D), jnp.float32),
          jax.ShapeDtypeStruct((N, D), jnp.float32),
      ],
  )

  Ws = jnp.stack([W2, jnp.eye(D, dtype=jnp.float32)])
  bs = jnp.stack([b1.reshape(1, D), b2.reshape(1, D)])

  def layer(carry, wb):
    g, _ = carry
    W, b = wb
    S = scat(g.reshape(_NQ * N, _DH), src, dst, zeros_acc)
    h, gn = mid(S, S, g, dis, W, b)
    return (gn, h), None

  h0 = jnp.zeros((N, D), jnp.float32)
  (_, h2), _ = lax.scan(layer, (g1, h0), (Ws, bs))

  csum, csq = pl.pallas_call(
      _tc_stats,
      grid=(nb,),
      in_specs=[pl.BlockSpec((RB, D), row)],
      out_specs=[
          pl.BlockSpec((1, D), full),
          pl.BlockSpec((1, D), full),
      ],
      out_shape=[
          jax.ShapeDtypeStruct((1, D), jnp.float32),
          jax.ShapeDtypeStruct((1, D), jnp.float32),
      ],
  )(h2)

  out = pl.pallas_call(
      _make_tc_final(N),
      grid=(nb,),
      in_specs=[
          pl.BlockSpec((RB, D), row),
          pl.BlockSpec((1, D), full),
          pl.BlockSpec((1, D), full),
      ],
      out_specs=pl.BlockSpec((RB, D), row),
      out_shape=jax.ShapeDtypeStruct((N, D), jnp.float32),
  )(h2, csum, csq)

  return out


# speedup vs baseline: 8.4250x; 1.6238x over previous
import functools

import jax
import jax.numpy as jnp
from jax import lax
from jax.experimental import pallas as pl
from jax.experimental.pallas import tpu as pltpu
from jax.experimental.pallas import tpu_sc as plsc

_NC = 2
_NS = 16
_CH = 128
_NQ = 2
_DH = 128


@functools.lru_cache(maxsize=None)
def _make_deg_kernel(E, NPAD):
  EC = E // _NC
  nch = EC // _CH
  per = -(-nch // _NS)
  NZ = NPAD // _NS
  mesh = plsc.VectorSubcoreMesh(core_axis_name="c", subcore_axis_name="s")

  @functools.partial(
      pl.kernel, mesh=mesh,
      out_type=jax.ShapeDtypeStruct((_NC * NPAD,), jnp.float32),
      scratch_types=[
          pltpu.VMEM((_CH,), jnp.int32),
          pltpu.VMEM((_CH,), jnp.float32),
          pltpu.VMEM((NZ,), jnp.float32),
          pltpu.VMEM_SHARED((NPAD,), jnp.float32),
      ],
  )
  def deg_k(dst_hbm, zeros_hbm, out_hbm, idxv, onesv, zbuf, acc):
    c = lax.axis_index("c")
    s = lax.axis_index("s")
    pltpu.sync_copy(zeros_hbm, zbuf)
    pltpu.sync_copy(zbuf, acc.at[pl.ds(s * NZ, NZ)])
    for i in range(_CH // 16):
      onesv[pl.ds(i * 16, 16)] = jnp.full((16,), 1.0, jnp.float32)
    plsc.subcore_barrier()

    def body(k, carry):
      ch = s + k * _NS

      @pl.when(ch < nch)
      def _():
        base = c * EC + ch * _CH
        pltpu.sync_copy(dst_hbm.at[pl.ds(base, _CH)], idxv)
        pltpu.sync_copy(onesv, acc.at[idxv], add=True)

      return carry

    lax.fori_loop(0, per, body, 0)
    plsc.subcore_barrier()
    pltpu.sync_copy(acc.at[pl.ds(s * NZ, NZ)], zbuf)
    pltpu.sync_copy(zbuf, out_hbm.at[pl.ds(c * NPAD + s * NZ, NZ)])

  return deg_k


_NB = 3
_BCH = 6


@functools.lru_cache(maxsize=None)
def _make_scatter_kernel(E, N):
  nch = E // _CH
  cpt = nch // _NS // _BCH * _BCH
  nkb = cpt // _BCH
  tail = nch - cpt * _NS
  BLK = _BCH * _CH
  HALF = N // _NC
  ACC = HALF + 8
  NRA = (HALF // _NS + 7) // 8 * 8
  NRL = HALF - (_NS - 1) * NRA
  NZA = (ACC // _NS + 7) // 8 * 8
  NZL = ACC - (_NS - 1) * NZA
  mesh = plsc.VectorSubcoreMesh(core_axis_name="c", subcore_axis_name="s")

  @functools.partial(
      pl.kernel, mesh=mesh,
      out_type=jax.ShapeDtypeStruct((_NQ * N, _DH), jnp.float32),
      scratch_types=[
          pltpu.VMEM((BLK,), jnp.int32),
          pltpu.VMEM((BLK,), jnp.int32),
      ] + [pltpu.VMEM((_CH,), jnp.int32) for _ in range(2 * _NB)] + [
      ] + [pltpu.VMEM((_CH, _DH), jnp.float32) for _ in range(_NB)] + [
          pltpu.VMEM((_CH, _DH), jnp.float32),
          pltpu.VMEM_SHARED((ACC, _DH), jnp.float32),
      ] + [pltpu.SemaphoreType.DMA for _ in range(2 * _NB)],
  )
  def scat_k(g_hbm, src_hbm, dst_hbm, zeros_hbm, out_hbm,
             srcblk, dstblk, *refs):
    gidx = refs[0:_NB]
    dstl = refs[_NB:2 * _NB]
    rows = refs[2 * _NB:3 * _NB]
    zbuf, acc = refs[3 * _NB:3 * _NB + 2]
    gsem = refs[3 * _NB + 2:4 * _NB + 2]
    ssem = refs[4 * _NB + 2:]
    c = lax.axis_index("c")
    s = lax.axis_index("s")
    last = s == _NS - 1
    lo = c * HALF
    pltpu.sync_copy(zeros_hbm, zbuf)

    def adjust(t, j, p):
      for q in range(_CH // 16):
        bs = pl.ds(j * _CH + q * 16, 16)
        sl = pl.ds(q * 16, 16)
        gidx[t][sl] = srcblk[bs] * _NQ + p
        dl = dstblk[bs] - lo
        ok = (dl >= 0) & (dl < HALF)
        dstl[t][sl] = jnp.where(ok, dl, HALF)

    def fire_gather(t):
      pltpu.async_copy(g_hbm.at[gidx[t]], rows[t], gsem[t])

    def wait_gather(t):
      pltpu.make_async_copy(g_hbm.at[gidx[t]], rows[t], gsem[t]).wait()

    def fire_scatter(t):
      pltpu.async_copy(rows[t], acc.at[dstl[t]], ssem[t], add=True)

    def wait_scatter(t):
      pltpu.make_async_copy(rows[t], acc.at[dstl[t]], ssem[t]).wait()

    def pieces(n):
      return [(o, min(_CH, n - o)) for o in range(0, n, _CH)]

    for p in range(_NQ):
      @pl.when(~last)
      def _():
        for o, n in pieces(NZA):
          pltpu.sync_copy(zbuf.at[pl.ds(0, n)], acc.at[pl.ds(s * NZA + o, n)])

      @pl.when(last)
      def _():
        for o, n in pieces(NZL):
          pltpu.sync_copy(zbuf.at[pl.ds(0, n)], acc.at[pl.ds(s * NZA + o, n)])

      plsc.subcore_barrier()

      ebase = s * (cpt * _CH)

      def body(kb, carry):
        blo = ebase + kb * BLK
        pltpu.sync_copy(src_hbm.at[pl.ds(blo, BLK)], srcblk)
        pltpu.sync_copy(dst_hbm.at[pl.ds(blo, BLK)], dstblk)
        for j in range(_BCH):
          t = j % _NB
          u = (j - 1) % _NB
          if j >= _NB:
            wait_scatter(t)
          else:
            @pl.when(kb > 0)
            def _():
              wait_scatter(t)
          adjust(t, j, p)
          fire_gather(t)
          if j >= 1:
            wait_gather(u)
            fire_scatter(u)
          else:
            @pl.when(kb > 0)
            def _():
              wait_gather(u)
              fire_scatter(u)
        return carry

      lax.fori_loop(0, nkb, body, 0)
      tl = (_BCH - 1) % _NB
      wait_gather(tl)
      fire_scatter(tl)
      for t in range(_NB):
        wait_scatter(t)

      if tail:
        @pl.when(s < tail)
        def _():
          base = pl.multiple_of((cpt * _NS + s) * _CH, 8)
          pltpu.sync_copy(src_hbm.at[pl.ds(base, _CH)],
                          srcblk.at[pl.ds(0, _CH)])
          pltpu.sync_copy(dst_hbm.at[pl.ds(base, _CH)],
                          dstblk.at[pl.ds(0, _CH)])
          adjust(0, 0, p)
          fire_gather(0)
          wait_gather(0)
          fire_scatter(0)
          wait_scatter(0)

      plsc.subcore_barrier()

      obase = pl.multiple_of(p * N + c * HALF + s * NRA, 8)

      @pl.when(~last)
      def _():
        for o, n in pieces(NRA):
          pltpu.sync_copy(acc.at[pl.ds(s * NRA + o, n)], zbuf.at[pl.ds(0, n)])
          pltpu.sync_copy(zbuf.at[pl.ds(0, n)], out_hbm.at[pl.ds(obase + o, n)])
        pltpu.sync_copy(zeros_hbm, zbuf)

      @pl.when(last)
      def _():
        for o, n in pieces(NRL):
          pltpu.sync_copy(acc.at[pl.ds(s * NRA + o, n)], zbuf.at[pl.ds(0, n)])
          pltpu.sync_copy(zbuf.at[pl.ds(0, n)], out_hbm.at[pl.ds(obase + o, n)])
        pltpu.sync_copy(zeros_hbm, zbuf)

  return scat_k


def _tc_first(x_ref, w_ref, d0_ref, d1_ref, g_ref, dis_ref):
  dis = lax.rsqrt(d0_ref[...] + d1_ref[...] + 1.0)
  z = jnp.dot(x_ref[...], w_ref[...], preferred_element_type=jnp.float32)
  g_ref[...] = z * dis
  dis_ref[...] = dis


def _tc_mid(s0_ref, s1_ref, g_ref, dis_ref, w_ref, b_ref, h_ref, gn_ref):
  dis = dis_ref[...]
  h = dis * (jnp.concatenate([s0_ref[...], s1_ref[...]], axis=1)
             + g_ref[...]) + b_ref[...]
  h_ref[...] = h
  z = jnp.dot(h, w_ref[...], preferred_element_type=jnp.float32)
  gn_ref[...] = z * dis


def _tc_stats(h_ref, sum_ref, sq_ref):
  i = pl.program_id(0)
  h = h_ref[...]
  ps = jnp.sum(h, axis=0, keepdims=True)
  pq = jnp.sum(h * h, axis=0, keepdims=True)

  @pl.when(i == 0)
  def _():
    sum_ref[...] = ps
    sq_ref[...] = pq

  @pl.when(i > 0)
  def _():
    sum_ref[...] += ps
    sq_ref[...] += pq


def _make_tc_final(N):
  def _tc_final(h_ref, sum_ref, sq_ref, out_ref):
    n = float(N)
    mean = sum_ref[...] / n
    var = (sq_ref[...] - sum_ref[...] * mean) / (n - 1.0)
    out_ref[...] = (h_ref[...] - mean) * lax.rsqrt(var)

  return _tc_final


def kernel(x, edge_index, W1, b1, W2, b2):
  N, D = x.shape
  E = edge_index.shape[1]
  RB = 1000
  nb = N // RB
  NPAD = ((N + 16 * 8 - 1) // (16 * 8)) * (16 * 8)

  src = edge_index[0]
  dst = edge_index[1]
  zeros_deg = jnp.zeros((NPAD // _NS,), jnp.float32)
  zeros_acc = jnp.zeros((128, _DH), jnp.float32)

  degp = _make_deg_kernel(E, NPAD)(dst, zeros_deg)
  d0 = degp[:N].reshape(N, 1)
  d1 = degp[NPAD:NPAD + N].reshape(N, 1)

  row = lambda i: (i, 0)
  full = lambda i: (0, 0)
  g1, dis = pl.pallas_call(
      _tc_first,
      grid=(nb,),
      in_specs=[
          pl.BlockSpec((RB, D), row),
          pl.BlockSpec((D, D), full),
          pl.BlockSpec((RB, 1), row),
          pl.BlockSpec((RB, 1), row),
      ],
      out_specs=[
          pl.BlockSpec((RB, D), row),
          pl.BlockSpec((RB, 1), row),
      ],
      out_shape=[
          jax.ShapeDtypeStruct((N, D), jnp.float32),
          jax.ShapeDtypeStruct((N, 1), jnp.float32),
      ],
  )(x, W1, d0, d1)

  scat = _make_scatter_kernel(E, N)

  quad = [lambda i, q=q: (q * nb + i, 0) for q in range(_NQ)]
  mid = pl.pallas_call(
      _tc_mid,
      grid=(nb,),
      in_specs=[pl.BlockSpec((RB, _DH), quad[q]) for q in range(_NQ)] + [
          pl.BlockSpec((RB, D), row),
          pl.BlockSpec((RB, 1), row),
          pl.BlockSpec((D, D), full),
          pl.BlockSpec((1, D), full),
      ],
      out_specs=[
          pl.BlockSpec((RB, D), row),
          pl.BlockSpec((RB, D), row),
      ],
      out_shape=[
          jax.ShapeDtypeStruct((N, D), jnp.float32),
          jax.ShapeDtypeStruct((N, D), jnp.float32),
      ],
  )

  Ws = jnp.stack([W2, jnp.eye(D, dtype=jnp.float32)])
  bs = jnp.stack([b1.reshape(1, D), b2.reshape(1, D)])

  def layer(carry, wb):
    g, _ = carry
    W, b = wb
    S = scat(g.reshape(_NQ * N, _DH), src, dst, zeros_acc)
    h, gn = mid(S, S, g, dis, W, b)
    return (gn, h), None

  h0 = jnp.zeros((N, D), jnp.float32)
  (_, h2), _ = lax.scan(layer, (g1, h0), (Ws, bs))

  csum, csq = pl.pallas_call(
      _tc_stats,
      grid=(nb,),
      in_specs=[pl.BlockSpec((RB, D), row)],
      out_specs=[
          pl.BlockSpec((1, D), full),
          pl.BlockSpec((1, D), full),
      ],
      out_shape=[
          jax.ShapeDtypeStruct((1, D), jnp.float32),
          jax.ShapeDtypeStruct((1, D), jnp.float32),
      ],
  )(h2)

  out = pl.pallas_call(
      _make_tc_final(N),
      grid=(nb,),
      in_specs=[
          pl.BlockSpec((RB, D), row),
          pl.BlockSpec((1, D), full),
          pl.BlockSpec((1, D), full),
      ],
      out_specs=pl.BlockSpec((RB, D), row),
      out_shape=jax.ShapeDtypeStruct((N, D), jnp.float32),
  )(h2, csum, csq)

  return out


# speedup vs baseline: 10.6681x; 1.2662x over previous
import functools

import jax
import jax.numpy as jnp
from jax import lax
from jax.experimental import pallas as pl
from jax.experimental.pallas import tpu as pltpu
from jax.experimental.pallas import tpu_sc as plsc

_NC = 2
_NS = 16
_CH = 128
_NQ = 2
_DH = 128


@functools.lru_cache(maxsize=None)
def _make_deg_kernel(E, NPAD):
  EC = E // _NC
  nch = EC // _CH
  per = -(-nch // _NS)
  NZ = NPAD // _NS
  mesh = plsc.VectorSubcoreMesh(core_axis_name="c", subcore_axis_name="s")

  @functools.partial(
      pl.kernel, mesh=mesh,
      out_type=jax.ShapeDtypeStruct((_NC * NPAD,), jnp.float32),
      scratch_types=[
          pltpu.VMEM((_CH,), jnp.int32),
          pltpu.VMEM((_CH,), jnp.float32),
          pltpu.VMEM((NZ,), jnp.float32),
          pltpu.VMEM_SHARED((NPAD,), jnp.float32),
      ],
  )
  def deg_k(dst_hbm, zeros_hbm, out_hbm, idxv, onesv, zbuf, acc):
    c = lax.axis_index("c")
    s = lax.axis_index("s")
    pltpu.sync_copy(zeros_hbm, zbuf)
    pltpu.sync_copy(zbuf, acc.at[pl.ds(s * NZ, NZ)])
    for i in range(_CH // 16):
      onesv[pl.ds(i * 16, 16)] = jnp.full((16,), 1.0, jnp.float32)
    plsc.subcore_barrier()

    def body(k, carry):
      ch = s + k * _NS

      @pl.when(ch < nch)
      def _():
        base = c * EC + ch * _CH
        pltpu.sync_copy(dst_hbm.at[pl.ds(base, _CH)], idxv)
        pltpu.sync_copy(onesv, acc.at[idxv], add=True)

      return carry

    lax.fori_loop(0, per, body, 0)
    plsc.subcore_barrier()
    pltpu.sync_copy(acc.at[pl.ds(s * NZ, NZ)], zbuf)
    pltpu.sync_copy(zbuf, out_hbm.at[pl.ds(c * NPAD + s * NZ, NZ)])

  return deg_k


_NB = 3
_BCH = 6


@functools.lru_cache(maxsize=None)
def _make_scatter_kernel(E, N):
  nch = E // _CH
  cpt = nch // _NS // _BCH * _BCH
  nkb = cpt // _BCH
  tail = nch - cpt * _NS
  BLK = _BCH * _CH
  HALF = N // _NC
  ACC = HALF + 64
  NRA = (HALF // _NS + 7) // 8 * 8
  NRL = HALF - (_NS - 1) * NRA
  NZA = (ACC // _NS + 7) // 8 * 8
  NZL = ACC - (_NS - 1) * NZA
  mesh = plsc.VectorSubcoreMesh(core_axis_name="c", subcore_axis_name="s")

  @functools.partial(
      pl.kernel, mesh=mesh,
      out_type=jax.ShapeDtypeStruct((_NQ * N, _DH), jnp.float32),
      scratch_types=[
          pltpu.VMEM((BLK,), jnp.int32),
          pltpu.VMEM((BLK,), jnp.int32),
      ] + [pltpu.VMEM((_CH,), jnp.int32) for _ in range(2 * _NB)] + [
      ] + [pltpu.VMEM((_CH, _DH), jnp.float32) for _ in range(_NB)] + [
          pltpu.VMEM((_CH, _DH), jnp.float32),
          pltpu.VMEM_SHARED((ACC, _DH), jnp.float32),
      ] + [pltpu.SemaphoreType.DMA for _ in range(2 * _NB)],
  )
  def scat_k(g_hbm, src_hbm, dst_hbm, zeros_hbm, out_hbm,
             srcblk, dstblk, *refs):
    gidx = refs[0:_NB]
    dstl = refs[_NB:2 * _NB]
    rows = refs[2 * _NB:3 * _NB]
    zbuf, acc = refs[3 * _NB:3 * _NB + 2]
    gsem = refs[3 * _NB + 2:4 * _NB + 2]
    ssem = refs[4 * _NB + 2:]
    c = lax.axis_index("c")
    s = lax.axis_index("s")
    last = s == _NS - 1
    lo = c * HALF
    pltpu.sync_copy(zeros_hbm, zbuf)

    iota16 = lax.iota(jnp.int32, 16)

    def adjust(t, j, p):
      for q in range(_CH // 16):
        bs = pl.ds(j * _CH + q * 16, 16)
        sl = pl.ds(q * 16, 16)
        gidx[t][sl] = srcblk[bs] * _NQ + p
        dl = dstblk[bs] - lo
        ok = (dl >= 0) & (dl < HALF)
        trash = HALF + (((q + j) * 16) & 63) + iota16
        dstl[t][sl] = jnp.where(ok, dl, trash)

    def fire_gather(t):
      pltpu.async_copy(g_hbm.at[gidx[t]], rows[t], gsem[t])

    def wait_gather(t):
      pltpu.make_async_copy(g_hbm.at[gidx[t]], rows[t], gsem[t]).wait()

    def fire_scatter(t):
      pltpu.async_copy(rows[t], acc.at[dstl[t]], ssem[t], add=True)

    def wait_scatter(t):
      pltpu.make_async_copy(rows[t], acc.at[dstl[t]], ssem[t]).wait()

    def pieces(n):
      return [(o, min(_CH, n - o)) for o in range(0, n, _CH)]

    for p in range(_NQ):
      @pl.when(~last)
      def _():
        for o, n in pieces(NZA):
          pltpu.sync_copy(zbuf.at[pl.ds(0, n)], acc.at[pl.ds(s * NZA + o, n)])

      @pl.when(last)
      def _():
        for o, n in pieces(NZL):
          pltpu.sync_copy(zbuf.at[pl.ds(0, n)], acc.at[pl.ds(s * NZA + o, n)])

      plsc.subcore_barrier()

      ebase = s * (cpt * _CH)

      def body(kb, carry):
        blo = ebase + kb * BLK
        pltpu.sync_copy(src_hbm.at[pl.ds(blo, BLK)], srcblk)
        pltpu.sync_copy(dst_hbm.at[pl.ds(blo, BLK)], dstblk)
        for j in range(_BCH):
          t = j % _NB
          u = (j - 2) % _NB
          if j >= _NB:
            wait_scatter(t)
          else:
            @pl.when(kb > 0)
            def _():
              wait_scatter(t)
          adjust(t, j, p)
          fire_gather(t)
          if j >= 2:
            wait_gather(u)
            fire_scatter(u)
          else:
            @pl.when(kb > 0)
            def _():
              wait_gather(u)
              fire_scatter(u)
        return carry

      lax.fori_loop(0, nkb, body, 0)
      for dj in (_BCH - 2, _BCH - 1):
        tl = dj % _NB
        wait_gather(tl)
        fire_scatter(tl)
      for t in range(_NB):
        wait_scatter(t)

      if tail:
        @pl.when(s < tail)
        def _():
          base = pl.multiple_of((cpt * _NS + s) * _CH, 8)
          pltpu.sync_copy(src_hbm.at[pl.ds(base, _CH)],
                          srcblk.at[pl.ds(0, _CH)])
          pltpu.sync_copy(dst_hbm.at[pl.ds(base, _CH)],
                          dstblk.at[pl.ds(0, _CH)])
          adjust(0, 0, p)
          fire_gather(0)
          wait_gather(0)
          fire_scatter(0)
          wait_scatter(0)

      plsc.subcore_barrier()

      obase = pl.multiple_of(p * N + c * HALF + s * NRA, 8)

      @pl.when(~last)
      def _():
        for o, n in pieces(NRA):
          pltpu.sync_copy(acc.at[pl.ds(s * NRA + o, n)], zbuf.at[pl.ds(0, n)])
          pltpu.sync_copy(zbuf.at[pl.ds(0, n)], out_hbm.at[pl.ds(obase + o, n)])
        pltpu.sync_copy(zeros_hbm, zbuf)

      @pl.when(last)
      def _():
        for o, n in pieces(NRL):
          pltpu.sync_copy(acc.at[pl.ds(s * NRA + o, n)], zbuf.at[pl.ds(0, n)])
          pltpu.sync_copy(zbuf.at[pl.ds(0, n)], out_hbm.at[pl.ds(obase + o, n)])
        pltpu.sync_copy(zeros_hbm, zbuf)

  return scat_k


def _tc_first(x_ref, w_ref, d0_ref, d1_ref, g_ref, dis_ref):
  dis = lax.rsqrt(d0_ref[...] + d1_ref[...] + 1.0)
  z = jnp.dot(x_ref[...], w_ref[...], preferred_element_type=jnp.float32)
  g_ref[...] = z * dis
  dis_ref[...] = dis


def _tc_mid(s0_ref, s1_ref, g_ref, dis_ref, w_ref, b_ref, h_ref, gn_ref):
  dis = dis_ref[...]
  h = dis * (jnp.concatenate([s0_ref[...], s1_ref[...]], axis=1)
             + g_ref[...]) + b_ref[...]
  h_ref[...] = h
  z = jnp.dot(h, w_ref[...], preferred_element_type=jnp.float32)
  gn_ref[...] = z * dis


def _tc_stats(h_ref, sum_ref, sq_ref):
  i = pl.program_id(0)
  h = h_ref[...]
  ps = jnp.sum(h, axis=0, keepdims=True)
  pq = jnp.sum(h * h, axis=0, keepdims=True)

  @pl.when(i == 0)
  def _():
    sum_ref[...] = ps
    sq_ref[...] = pq

  @pl.when(i > 0)
  def _():
    sum_ref[...] += ps
    sq_ref[...] += pq


def _make_tc_final(N):
  def _tc_final(h_ref, sum_ref, sq_ref, out_ref):
    n = float(N)
    mean = sum_ref[...] / n
    var = (sq_ref[...] - sum_ref[...] * mean) / (n - 1.0)
    out_ref[...] = (h_ref[...] - mean) * lax.rsqrt(var)

  return _tc_final


def kernel(x, edge_index, W1, b1, W2, b2):
  N, D = x.shape
  E = edge_index.shape[1]
  RB = 1000
  nb = N // RB
  NPAD = ((N + 16 * 8 - 1) // (16 * 8)) * (16 * 8)

  src = edge_index[0]
  dst = edge_index[1]
  zeros_deg = jnp.zeros((NPAD // _NS,), jnp.float32)
  zeros_acc = jnp.zeros((128, _DH), jnp.float32)

  degp = _make_deg_kernel(E, NPAD)(dst, zeros_deg)
  d0 = degp[:N].reshape(N, 1)
  d1 = degp[NPAD:NPAD + N].reshape(N, 1)

  row = lambda i: (i, 0)
  full = lambda i: (0, 0)
  g1, dis = pl.pallas_call(
      _tc_first,
      grid=(nb,),
      in_specs=[
          pl.BlockSpec((RB, D), row),
          pl.BlockSpec((D, D), full),
          pl.BlockSpec((RB, 1), row),
          pl.BlockSpec((RB, 1), row),
      ],
      out_specs=[
          pl.BlockSpec((RB, D), row),
          pl.BlockSpec((RB, 1), row),
      ],
      out_shape=[
          jax.ShapeDtypeStruct((N, D), jnp.float32),
          jax.ShapeDtypeStruct((N, 1), jnp.float32),
      ],
  )(x, W1, d0, d1)

  scat = _make_scatter_kernel(E, N)

  quad = [lambda i, q=q: (q * nb + i, 0) for q in range(_NQ)]
  mid = pl.pallas_call(
      _tc_mid,
      grid=(nb,),
      in_specs=[pl.BlockSpec((RB, _DH), quad[q]) for q in range(_NQ)] + [
          pl.BlockSpec((RB, D), row),
          pl.BlockSpec((RB, 1), row),
          pl.BlockSpec((D, D), full),
          pl.BlockSpec((1, D), full),
      ],
      out_specs=[
          pl.BlockSpec((RB, D), row),
          pl.BlockSpec((RB, D), row),
      ],
      out_shape=[
          jax.ShapeDtypeStruct((N, D), jnp.float32),
          jax.ShapeDtypeStruct((N, D), jnp.float32),
      ],
  )

  Ws = jnp.stack([W2, jnp.eye(D, dtype=jnp.float32)])
  bs = jnp.stack([b1.reshape(1, D), b2.reshape(1, D)])

  def layer(carry, wb):
    g, _ = carry
    W, b = wb
    S = scat(g.reshape(_NQ * N, _DH), src, dst, zeros_acc)
    h, gn = mid(S, S, g, dis, W, b)
    return (gn, h), None

  h0 = jnp.zeros((N, D), jnp.float32)
  (_, h2), _ = lax.scan(layer, (g1, h0), (Ws, bs))

  csum, csq = pl.pallas_call(
      _tc_stats,
      grid=(nb,),
      in_specs=[pl.BlockSpec((RB, D), row)],
      out_specs=[
          pl.BlockSpec((1, D), full),
          pl.BlockSpec((1, D), full),
      ],
      out_shape=[
          jax.ShapeDtypeStruct((1, D), jnp.float32),
          jax.ShapeDtypeStruct((1, D), jnp.float32),
      ],
  )(h2)

  out = pl.pallas_call(
      _make_tc_final(N),
      grid=(nb,),
      in_specs=[
          pl.BlockSpec((RB, D), row),
          pl.BlockSpec((1, D), full),
          pl.BlockSpec((1, D), full),
      ],
      out_specs=pl.BlockSpec((RB, D), row),
      out_shape=jax.ShapeDtypeStruct((N, D), jnp.float32),
  )(h2, csum, csq)

  return out


# speedup vs baseline: 10.7098x; 1.0039x over previous
import functools

import jax
import jax.numpy as jnp
from jax import lax
from jax.experimental import pallas as pl
from jax.experimental.pallas import tpu as pltpu
from jax.experimental.pallas import tpu_sc as plsc

_NC = 2
_NS = 16
_CH = 128
_NQ = 2
_DH = 128


@functools.lru_cache(maxsize=None)
def _make_deg_kernel(E, NPAD):
  EC = E // _NC
  nch = EC // _CH
  per = -(-nch // _NS)
  NZ = NPAD // _NS
  mesh = plsc.VectorSubcoreMesh(core_axis_name="c", subcore_axis_name="s")

  @functools.partial(
      pl.kernel, mesh=mesh,
      out_type=jax.ShapeDtypeStruct((_NC * NPAD,), jnp.float32),
      scratch_types=[
          pltpu.VMEM((_CH,), jnp.int32),
          pltpu.VMEM((_CH,), jnp.float32),
          pltpu.VMEM((NZ,), jnp.float32),
          pltpu.VMEM_SHARED((NPAD,), jnp.float32),
      ],
  )
  def deg_k(dst_hbm, zeros_hbm, out_hbm, idxv, onesv, zbuf, acc):
    c = lax.axis_index("c")
    s = lax.axis_index("s")
    pltpu.sync_copy(zeros_hbm, zbuf)
    pltpu.sync_copy(zbuf, acc.at[pl.ds(s * NZ, NZ)])
    for i in range(_CH // 16):
      onesv[pl.ds(i * 16, 16)] = jnp.full((16,), 1.0, jnp.float32)
    plsc.subcore_barrier()

    def body(k, carry):
      ch = s + k * _NS

      @pl.when(ch < nch)
      def _():
        base = c * EC + ch * _CH
        pltpu.sync_copy(dst_hbm.at[pl.ds(base, _CH)], idxv)
        pltpu.sync_copy(onesv, acc.at[idxv], add=True)

      return carry

    lax.fori_loop(0, per, body, 0)
    plsc.subcore_barrier()
    pltpu.sync_copy(acc.at[pl.ds(s * NZ, NZ)], zbuf)
    pltpu.sync_copy(zbuf, out_hbm.at[pl.ds(c * NPAD + s * NZ, NZ)])

  return deg_k


_NB = 3
_BCH = 6


@functools.lru_cache(maxsize=None)
def _make_scatter_kernel(E, N):
  nch = E // _CH
  cpt = nch // _NS // _BCH * _BCH
  nkb = cpt // _BCH
  tail = nch - cpt * _NS
  BLK = _BCH * _CH
  HALF = N // _NC
  ACC = HALF + 256
  NRA = (HALF // _NS + 7) // 8 * 8
  NRL = HALF - (_NS - 1) * NRA
  NZA = (ACC // _NS + 7) // 8 * 8
  NZL = ACC - (_NS - 1) * NZA
  mesh = plsc.VectorSubcoreMesh(core_axis_name="c", subcore_axis_name="s")

  @functools.partial(
      pl.kernel, mesh=mesh,
      out_type=jax.ShapeDtypeStruct((_NQ * N, _DH), jnp.float32),
      scratch_types=[
          pltpu.VMEM((BLK,), jnp.int32),
          pltpu.VMEM((BLK,), jnp.int32),
      ] + [pltpu.VMEM((_CH,), jnp.int32) for _ in range(2 * _NB)] + [
      ] + [pltpu.VMEM((_CH, _DH), jnp.float32) for _ in range(_NB)] + [
          pltpu.VMEM((_CH, _DH), jnp.float32),
          pltpu.VMEM_SHARED((ACC, _DH), jnp.float32),
      ] + [pltpu.SemaphoreType.DMA for _ in range(2 * _NB)],
  )
  def scat_k(g_hbm, src_hbm, dst_hbm, zeros_hbm, out_hbm,
             srcblk, dstblk, *refs):
    gidx = refs[0:_NB]
    dstl = refs[_NB:2 * _NB]
    rows = refs[2 * _NB:3 * _NB]
    zbuf, acc = refs[3 * _NB:3 * _NB + 2]
    gsem = refs[3 * _NB + 2:4 * _NB + 2]
    ssem = refs[4 * _NB + 2:]
    c = lax.axis_index("c")
    s = lax.axis_index("s")
    last = s == _NS - 1
    lo = c * HALF
    pltpu.sync_copy(zeros_hbm, zbuf)

    iota16 = lax.iota(jnp.int32, 16)

    def adjust(t, j, p):
      for q in range(_CH // 16):
        bs = pl.ds(j * _CH + q * 16, 16)
        sl = pl.ds(q * 16, 16)
        gidx[t][sl] = srcblk[bs] * _NQ + p
        dl = dstblk[bs] - lo
        ok = (dl >= 0) & (dl < HALF)
        trash = HALF + ((s * 16 + (q + j) * 16) & 255) + iota16
        dstl[t][sl] = jnp.where(ok, dl, trash)

    def fire_gather(t):
      pltpu.async_copy(g_hbm.at[gidx[t]], rows[t], gsem[t])

    def wait_gather(t):
      pltpu.make_async_copy(g_hbm.at[gidx[t]], rows[t], gsem[t]).wait()

    def fire_scatter(t):
      pltpu.async_copy(rows[t], acc.at[dstl[t]], ssem[t], add=True)

    def wait_scatter(t):
      pltpu.make_async_copy(rows[t], acc.at[dstl[t]], ssem[t]).wait()

    def pieces(n):
      return [(o, min(_CH, n - o)) for o in range(0, n, _CH)]

    for p in range(_NQ):
      @pl.when(~last)
      def _():
        for o, n in pieces(NZA):
          pltpu.sync_copy(zbuf.at[pl.ds(0, n)], acc.at[pl.ds(s * NZA + o, n)])

      @pl.when(last)
      def _():
        for o, n in pieces(NZL):
          pltpu.sync_copy(zbuf.at[pl.ds(0, n)], acc.at[pl.ds(s * NZA + o, n)])

      plsc.subcore_barrier()

      ebase = s * (cpt * _CH)

      def body(kb, carry):
        blo = ebase + kb * BLK
        pltpu.sync_copy(src_hbm.at[pl.ds(blo, BLK)], srcblk)
        pltpu.sync_copy(dst_hbm.at[pl.ds(blo, BLK)], dstblk)
        for j in range(_BCH):
          t = j % _NB
          u = (j - 2) % _NB
          if j >= _NB:
            wait_scatter(t)
          else:
            @pl.when(kb > 0)
            def _():
              wait_scatter(t)
          adjust(t, j, p)
          fire_gather(t)
          if j >= 2:
            wait_gather(u)
            fire_scatter(u)
          else:
            @pl.when(kb > 0)
            def _():
              wait_gather(u)
              fire_scatter(u)
        return carry

      lax.fori_loop(0, nkb, body, 0)
      for dj in (_BCH - 2, _BCH - 1):
        tl = dj % _NB
        wait_gather(tl)
        fire_scatter(tl)
      for t in range(_NB):
        wait_scatter(t)

      if tail:
        @pl.when(s < tail)
        def _():
          base = pl.multiple_of((cpt * _NS + s) * _CH, 8)
          pltpu.sync_copy(src_hbm.at[pl.ds(base, _CH)],
                          srcblk.at[pl.ds(0, _CH)])
          pltpu.sync_copy(dst_hbm.at[pl.ds(base, _CH)],
                          dstblk.at[pl.ds(0, _CH)])
          adjust(0, 0, p)
          fire_gather(0)
          wait_gather(0)
          fire_scatter(0)
          wait_scatter(0)

      plsc.subcore_barrier()

      obase = pl.multiple_of(p * N + c * HALF + s * NRA, 8)

      @pl.when(~last)
      def _():
        for o, n in pieces(NRA):
          pltpu.sync_copy(acc.at[pl.ds(s * NRA + o, n)], zbuf.at[pl.ds(0, n)])
          pltpu.sync_copy(zbuf.at[pl.ds(0, n)], out_hbm.at[pl.ds(obase + o, n)])
        pltpu.sync_copy(zeros_hbm, zbuf)

      @pl.when(last)
      def _():
        for o, n in pieces(NRL):
          pltpu.sync_copy(acc.at[pl.ds(s * NRA + o, n)], zbuf.at[pl.ds(0, n)])
          pltpu.sync_copy(zbuf.at[pl.ds(0, n)], out_hbm.at[pl.ds(obase + o, n)])
        pltpu.sync_copy(zeros_hbm, zbuf)

  return scat_k


def _tc_mm(x_ref, w_ref, z_ref):
  z_ref[...] = jnp.dot(x_ref[...], w_ref[...],
                       preferred_element_type=jnp.float32)


def _tc_scale(z_ref, d0_ref, d1_ref, g_ref, dis_ref):
  dis = lax.rsqrt(d0_ref[...] + d1_ref[...] + 1.0)
  g_ref[...] = z_ref[...] * dis
  dis_ref[...] = dis


def _tc_mid(s0_ref, s1_ref, g_ref, dis_ref, w_ref, b_ref,
            h_ref, gn_ref, sum_ref, sq_ref):
  i = pl.program_id(0)
  dis = dis_ref[...]
  h = dis * (jnp.concatenate([s0_ref[...], s1_ref[...]], axis=1)
             + g_ref[...]) + b_ref[...]
  h_ref[...] = h
  z = jnp.dot(h, w_ref[...], preferred_element_type=jnp.float32)
  gn_ref[...] = z * dis
  ps = jnp.sum(h, axis=0, keepdims=True)
  pq = jnp.sum(h * h, axis=0, keepdims=True)

  @pl.when(i == 0)
  def _():
    sum_ref[...] = ps
    sq_ref[...] = pq

  @pl.when(i > 0)
  def _():
    sum_ref[...] += ps
    sq_ref[...] += pq


def _make_tc_final(N):
  def _tc_final(h_ref, sum_ref, sq_ref, out_ref):
    n = float(N)
    mean = sum_ref[...] / n
    var = (sq_ref[...] - sum_ref[...] * mean) / (n - 1.0)
    out_ref[...] = (h_ref[...] - mean) * lax.rsqrt(var)

  return _tc_final


def kernel(x, edge_index, W1, b1, W2, b2):
  N, D = x.shape
  E = edge_index.shape[1]
  RB = 1000
  nb = N // RB
  NPAD = ((N + 16 * 8 - 1) // (16 * 8)) * (16 * 8)

  src = edge_index[0]
  dst = edge_index[1]
  zeros_deg = jnp.zeros((NPAD // _NS,), jnp.float32)
  zeros_acc = jnp.zeros((128, _DH), jnp.float32)

  degp = _make_deg_kernel(E, NPAD)(dst, zeros_deg)
  d0 = degp[:N].reshape(N, 1)
  d1 = degp[NPAD:NPAD + N].reshape(N, 1)

  row = lambda i: (i, 0)
  full = lambda i: (0, 0)
  z1 = pl.pallas_call(
      _tc_mm,
      grid=(nb,),
      in_specs=[
          pl.BlockSpec((RB, D), row),
          pl.BlockSpec((D, D), full),
      ],
      out_specs=pl.BlockSpec((RB, D), row),
      out_shape=jax.ShapeDtypeStruct((N, D), jnp.float32),
  )(x, W1)

  g1, dis = pl.pallas_call(
      _tc_scale,
      grid=(nb,),
      in_specs=[
          pl.BlockSpec((RB, D), row),
          pl.BlockSpec((RB, 1), row),
          pl.BlockSpec((RB, 1), row),
      ],
      out_specs=[
          pl.BlockSpec((RB, D), row),
          pl.BlockSpec((RB, 1), row),
      ],
      out_shape=[
          jax.ShapeDtypeStruct((N, D), jnp.float32),
          jax.ShapeDtypeStruct((N, 1), jnp.float32),
      ],
  )(z1, d0, d1)

  scat = _make_scatter_kernel(E, N)

  rowa = lambda i: (i, 0)
  rowb = lambda i: (i + nb, 0)
  mid = pl.pallas_call(
      _tc_mid,
      grid=(nb,),
      in_specs=[
          pl.BlockSpec((RB, _DH), rowa),
          pl.BlockSpec((RB, _DH), rowb),
          pl.BlockSpec((RB, D), row),
          pl.BlockSpec((RB, 1), row),
          pl.BlockSpec((D, D), full),
          pl.BlockSpec((1, D), full),
      ],
      out_specs=[
          pl.BlockSpec((RB, D), row),
          pl.BlockSpec((RB, D), row),
          pl.BlockSpec((1, D), full),
          pl.BlockSpec((1, D), full),
      ],
      out_shape=[
          jax.ShapeDtypeStruct((N, D), jnp.float32),
          jax.ShapeDtypeStruct((N, D), jnp.float32),
          jax.ShapeDtypeStruct((1, D), jnp.float32),
          jax.ShapeDtypeStruct((1, D), jnp.float32),
      ],
  )

  Ws = jnp.stack([W2, jnp.eye(D, dtype=jnp.float32)])
  bs = jnp.stack([b1.reshape(1, D), b2.reshape(1, D)])

  def layer(carry, wb):
    g, _, _, _ = carry
    W, b = wb
    S = scat(g.reshape(_NQ * N, _DH), src, dst, zeros_acc)
    h, gn, cs, cq = mid(S, S, g, dis, W, b)
    return (gn, h, cs, cq), None

  h0 = jnp.zeros((N, D), jnp.float32)
  c0 = jnp.zeros((1, D), jnp.float32)
  (_, h2, csum, csq), _ = lax.scan(layer, (g1, h0, c0, c0), (Ws, bs))

  out = pl.pallas_call(
      _make_tc_final(N),
      grid=(nb,),
      in_specs=[
          pl.BlockSpec((RB, D), row),
          pl.BlockSpec((1, D), full),
          pl.BlockSpec((1, D), full),
      ],
      out_specs=pl.BlockSpec((RB, D), row),
      out_shape=jax.ShapeDtypeStruct((N, D), jnp.float32),
  )(h2, csum, csq)

  return out


# speedup vs baseline: 10.8503x; 1.0131x over previous
import functools

import jax
import jax.numpy as jnp
from jax import lax
from jax.experimental import pallas as pl
from jax.experimental.pallas import tpu as pltpu
from jax.experimental.pallas import tpu_sc as plsc

_NC = 2
_NS = 16
_CH = 128
_NQ = 2
_DH = 128


@functools.lru_cache(maxsize=None)
def _make_deg_kernel(E, NPAD):
  EC = E // _NC
  nch = EC // _CH
  per = -(-nch // _NS)
  NZ = NPAD // _NS
  mesh = plsc.VectorSubcoreMesh(core_axis_name="c", subcore_axis_name="s")

  @functools.partial(
      pl.kernel, mesh=mesh,
      out_type=jax.ShapeDtypeStruct((_NC * NPAD,), jnp.float32),
      scratch_types=[
          pltpu.VMEM((_CH,), jnp.int32),
          pltpu.VMEM((_CH,), jnp.float32),
          pltpu.VMEM((NZ,), jnp.float32),
          pltpu.VMEM_SHARED((NPAD,), jnp.float32),
      ],
  )
  def deg_k(dst_hbm, zeros_hbm, out_hbm, idxv, onesv, zbuf, acc):
    c = lax.axis_index("c")
    s = lax.axis_index("s")
    pltpu.sync_copy(zeros_hbm, zbuf)
    pltpu.sync_copy(zbuf, acc.at[pl.ds(s * NZ, NZ)])
    for i in range(_CH // 16):
      onesv[pl.ds(i * 16, 16)] = jnp.full((16,), 1.0, jnp.float32)
    plsc.subcore_barrier()

    def body(k, carry):
      ch = s + k * _NS

      @pl.when(ch < nch)
      def _():
        base = c * EC + ch * _CH
        pltpu.sync_copy(dst_hbm.at[pl.ds(base, _CH)], idxv)
        pltpu.sync_copy(onesv, acc.at[idxv], add=True)

      return carry

    lax.fori_loop(0, per, body, 0)
    plsc.subcore_barrier()
    pltpu.sync_copy(acc.at[pl.ds(s * NZ, NZ)], zbuf)
    pltpu.sync_copy(zbuf, out_hbm.at[pl.ds(c * NPAD + s * NZ, NZ)])

  return deg_k


_NB = 3
_BCH = 6


@functools.lru_cache(maxsize=None)
def _make_scatter_kernel(E, N):
  nch = E // _CH
  cpt = nch // _NS // _BCH * _BCH
  nkb = cpt // _BCH
  tail = nch - cpt * _NS
  BLK = _BCH * _CH
  HALF = N // _NC
  ACC = HALF + 256
  NRA = (HALF // _NS + 7) // 8 * 8
  NRL = HALF - (_NS - 1) * NRA
  NZA = (ACC // _NS + 7) // 8 * 8
  NZL = ACC - (_NS - 1) * NZA
  mesh = plsc.VectorSubcoreMesh(core_axis_name="c", subcore_axis_name="s")

  @functools.partial(
      pl.kernel, mesh=mesh,
      out_type=jax.ShapeDtypeStruct((_NQ * N, _DH), jnp.float32),
      scratch_types=[
          pltpu.VMEM((BLK,), jnp.int32),
          pltpu.VMEM((BLK,), jnp.int32),
          pltpu.VMEM((BLK,), jnp.int32),
          pltpu.VMEM((BLK,), jnp.int32),
      ] + [pltpu.VMEM((_CH,), jnp.int32) for _ in range(2 * _NB)] + [
      ] + [pltpu.SemaphoreType.DMA, pltpu.SemaphoreType.DMA] + [
      ] + [pltpu.VMEM((_CH, _DH), jnp.float32) for _ in range(_NB)] + [
          pltpu.VMEM((_CH, _DH), jnp.float32),
          pltpu.VMEM_SHARED((ACC, _DH), jnp.float32),
      ] + [pltpu.SemaphoreType.DMA for _ in range(2 * _NB)],
  )
  def scat_k(g_hbm, src_hbm, dst_hbm, zeros_hbm, out_hbm,
             srcblk0, dstblk0, srcblk1, dstblk1, *refs):
    srcblks = (srcblk0, srcblk1)
    dstblks = (dstblk0, dstblk1)
    gidx = refs[0:_NB]
    dstl = refs[_NB:2 * _NB]
    bsem = refs[2 * _NB:2 * _NB + 2]
    rows = refs[2 * _NB + 2:3 * _NB + 2]
    zbuf, acc = refs[3 * _NB + 2:3 * _NB + 4]
    gsem = refs[3 * _NB + 4:4 * _NB + 4]
    ssem = refs[4 * _NB + 4:]
    c = lax.axis_index("c")
    s = lax.axis_index("s")
    last = s == _NS - 1
    lo = c * HALF
    ebase = s * (cpt * _CH)
    pltpu.sync_copy(zeros_hbm, zbuf)

    iota16 = lax.iota(jnp.int32, 16)

    def adjust(t, j, p, par):
      for q in range(_CH // 16):
        bs = pl.ds(j * _CH + q * 16, 16)
        sl = pl.ds(q * 16, 16)
        gidx[t][sl] = srcblks[par][bs] * _NQ + p
        dl = dstblks[par][bs] - lo
        ok = (dl >= 0) & (dl < HALF)
        trash = HALF + ((s * 16 + (q + j) * 16) & 255) + iota16
        dstl[t][sl] = jnp.where(ok, dl, trash)

    def fire_blk(b, par):
      blo = ebase + b * BLK
      pltpu.async_copy(src_hbm.at[pl.ds(blo, BLK)], srcblks[par], bsem[par])
      pltpu.async_copy(dst_hbm.at[pl.ds(blo, BLK)], dstblks[par], bsem[par])

    def wait_blk(b, par):
      blo = ebase + b * BLK
      pltpu.make_async_copy(src_hbm.at[pl.ds(blo, BLK)], srcblks[par],
                            bsem[par]).wait()
      pltpu.make_async_copy(dst_hbm.at[pl.ds(blo, BLK)], dstblks[par],
                            bsem[par]).wait()

    def fire_gather(t):
      pltpu.async_copy(g_hbm.at[gidx[t]], rows[t], gsem[t])

    def wait_gather(t):
      pltpu.make_async_copy(g_hbm.at[gidx[t]], rows[t], gsem[t]).wait()

    def fire_scatter(t):
      pltpu.async_copy(rows[t], acc.at[dstl[t]], ssem[t], add=True)

    def wait_scatter(t):
      pltpu.make_async_copy(rows[t], acc.at[dstl[t]], ssem[t]).wait()

    def pieces(n):
      return [(o, min(_CH, n - o)) for o in range(0, n, _CH)]

    for p in range(_NQ):
      @pl.when(~last)
      def _():
        for o, n in pieces(NZA):
          pltpu.sync_copy(zbuf.at[pl.ds(0, n)], acc.at[pl.ds(s * NZA + o, n)])

      @pl.when(last)
      def _():
        for o, n in pieces(NZL):
          pltpu.sync_copy(zbuf.at[pl.ds(0, n)], acc.at[pl.ds(s * NZA + o, n)])

      plsc.subcore_barrier()

      fire_blk(0, 0)

      def body(kb2, carry):
        for par in range(2):
          b = kb2 * 2 + par

          @pl.when(b < nkb)
          def _():
            wait_blk(b, par)

            @pl.when(b + 1 < nkb)
            def _():
              fire_blk(b + 1, 1 - par)

            for j in range(_BCH):
              t = j % _NB
              u = (j - 2) % _NB
              if j >= _NB:
                wait_scatter(t)
              else:
                @pl.when(b > 0)
                def _():
                  wait_scatter(t)
              adjust(t, j, p, par)
              fire_gather(t)
              if j >= 2:
                wait_gather(u)
                fire_scatter(u)
              else:
                @pl.when(b > 0)
                def _():
                  wait_gather(u)
                  fire_scatter(u)
        return carry

      lax.fori_loop(0, (nkb + 1) // 2, body, 0)
      for dj in (_BCH - 2, _BCH - 1):
        tl = dj % _NB
        wait_gather(tl)
        fire_scatter(tl)
      for t in range(_NB):
        wait_scatter(t)

      if tail:
        @pl.when(s < tail)
        def _():
          base = pl.multiple_of((cpt * _NS + s) * _CH, 8)
          pltpu.sync_copy(src_hbm.at[pl.ds(base, _CH)],
                          srcblk0.at[pl.ds(0, _CH)])
          pltpu.sync_copy(dst_hbm.at[pl.ds(base, _CH)],
                          dstblk0.at[pl.ds(0, _CH)])
          adjust(0, 0, p, 0)
          fire_gather(0)
          wait_gather(0)
          fire_scatter(0)
          wait_scatter(0)

      plsc.subcore_barrier()

      obase = pl.multiple_of(p * N + c * HALF + s * NRA, 8)

      @pl.when(~last)
      def _():
        for o, n in pieces(NRA):
          pltpu.sync_copy(acc.at[pl.ds(s * NRA + o, n)], zbuf.at[pl.ds(0, n)])
          pltpu.sync_copy(zbuf.at[pl.ds(0, n)], out_hbm.at[pl.ds(obase + o, n)])
        pltpu.sync_copy(zeros_hbm, zbuf)

      @pl.when(last)
      def _():
        for o, n in pieces(NRL):
          pltpu.sync_copy(acc.at[pl.ds(s * NRA + o, n)], zbuf.at[pl.ds(0, n)])
          pltpu.sync_copy(zbuf.at[pl.ds(0, n)], out_hbm.at[pl.ds(obase + o, n)])
        pltpu.sync_copy(zeros_hbm, zbuf)

  return scat_k


def _tc_mm(x_ref, w_ref, z_ref):
  z_ref[...] = jnp.dot(x_ref[...], w_ref[...],
                       preferred_element_type=jnp.float32)


def _tc_scale(z_ref, d0_ref, d1_ref, g_ref, dis_ref):
  dis = lax.rsqrt(d0_ref[...] + d1_ref[...] + 1.0)
  g_ref[...] = z_ref[...] * dis
  dis_ref[...] = dis


def _tc_mid(s0_ref, s1_ref, g_ref, dis_ref, w_ref, b_ref,
            h_ref, gn_ref, sum_ref, sq_ref):
  i = pl.program_id(0)
  dis = dis_ref[...]
  h = dis * (jnp.concatenate([s0_ref[...], s1_ref[...]], axis=1)
             + g_ref[...]) + b_ref[...]
  h_ref[...] = h
  z = jnp.dot(h, w_ref[...], preferred_element_type=jnp.float32)
  gn_ref[...] = z * dis
  ps = jnp.sum(h, axis=0, keepdims=True)
  pq = jnp.sum(h * h, axis=0, keepdims=True)

  @pl.when(i == 0)
  def _():
    sum_ref[...] = ps
    sq_ref[...] = pq

  @pl.when(i > 0)
  def _():
    sum_ref[...] += ps
    sq_ref[...] += pq


def _tc_last(s0_ref, s1_ref, g_ref, dis_ref, b_ref,
             h_ref, sum_ref, sq_ref):
  i = pl.program_id(0)
  h = dis_ref[...] * (jnp.concatenate([s0_ref[...], s1_ref[...]], axis=1)
                      + g_ref[...]) + b_ref[...]
  h_ref[...] = h
  ps = jnp.sum(h, axis=0, keepdims=True)
  pq = jnp.sum(h * h, axis=0, keepdims=True)

  @pl.when(i == 0)
  def _():
    sum_ref[...] = ps
    sq_ref[...] = pq

  @pl.when(i > 0)
  def _():
    sum_ref[...] += ps
    sq_ref[...] += pq


def _make_tc_final(N):
  def _tc_final(h_ref, sum_ref, sq_ref, out_ref):
    n = float(N)
    mean = sum_ref[...] / n
    var = (sq_ref[...] - sum_ref[...] * mean) / (n - 1.0)
    out_ref[...] = (h_ref[...] - mean) * lax.rsqrt(var)

  return _tc_final


def kernel(x, edge_index, W1, b1, W2, b2):
  N, D = x.shape
  E = edge_index.shape[1]
  RB = 1000
  nb = N // RB
  NPAD = ((N + 16 * 8 - 1) // (16 * 8)) * (16 * 8)

  src = edge_index[0]
  dst = edge_index[1]
  zeros_deg = jnp.zeros((NPAD // _NS,), jnp.float32)
  zeros_acc = jnp.zeros((128, _DH), jnp.float32)

  degp = _make_deg_kernel(E, NPAD)(dst, zeros_deg)
  d0 = degp[:N].reshape(N, 1)
  d1 = degp[NPAD:NPAD + N].reshape(N, 1)

  row = lambda i: (i, 0)
  full = lambda i: (0, 0)
  z1 = pl.pallas_call(
      _tc_mm,
      grid=(nb,),
      in_specs=[
          pl.BlockSpec((RB, D), row),
          pl.BlockSpec((D, D), full),
      ],
      out_specs=pl.BlockSpec((RB, D), row),
      out_shape=jax.ShapeDtypeStruct((N, D), jnp.float32),
  )(x, W1)

  g1, dis = pl.pallas_call(
      _tc_scale,
      grid=(nb,),
      in_specs=[
          pl.BlockSpec((RB, D), row),
          pl.BlockSpec((RB, 1), row),
          pl.BlockSpec((RB, 1), row),
      ],
      out_specs=[
          pl.BlockSpec((RB, D), row),
          pl.BlockSpec((RB, 1), row),
      ],
      out_shape=[
          jax.ShapeDtypeStruct((N, D), jnp.float32),
          jax.ShapeDtypeStruct((N, 1), jnp.float32),
      ],
  )(z1, d0, d1)

  scat = _make_scatter_kernel(E, N)

  rowa = lambda i: (i, 0)
  rowb = lambda i: (i + nb, 0)
  mid = pl.pallas_call(
      _tc_mid,
      grid=(nb,),
      in_specs=[
          pl.BlockSpec((RB, _DH), rowa),
          pl.BlockSpec((RB, _DH), rowb),
          pl.BlockSpec((RB, D), row),
          pl.BlockSpec((RB, 1), row),
          pl.BlockSpec((D, D), full),
          pl.BlockSpec((1, D), full),
      ],
      out_specs=[
          pl.BlockSpec((RB, D), row),
          pl.BlockSpec((RB, D), row),
          pl.BlockSpec((1, D), full),
          pl.BlockSpec((1, D), full),
      ],
      out_shape=[
          jax.ShapeDtypeStruct((N, D), jnp.float32),
          jax.ShapeDtypeStruct((N, D), jnp.float32),
          jax.ShapeDtypeStruct((1, D), jnp.float32),
          jax.ShapeDtypeStruct((1, D), jnp.float32),
      ],
  )

  lastk = pl.pallas_call(
      _tc_last,
      grid=(nb,),
      in_specs=[
          pl.BlockSpec((RB, _DH), rowa),
          pl.BlockSpec((RB, _DH), rowb),
          pl.BlockSpec((RB, D), row),
          pl.BlockSpec((RB, 1), row),
          pl.BlockSpec((1, D), full),
      ],
      out_specs=[
          pl.BlockSpec((RB, D), row),
          pl.BlockSpec((1, D), full),
          pl.BlockSpec((1, D), full),
      ],
      out_shape=[
          jax.ShapeDtypeStruct((N, D), jnp.float32),
          jax.ShapeDtypeStruct((1, D), jnp.float32),
          jax.ShapeDtypeStruct((1, D), jnp.float32),
      ],
  )

  Ws = jnp.stack([W2, W2])
  bs = jnp.stack([b1.reshape(1, D), b2.reshape(1, D)])
  firsts = jnp.array([1, 0], jnp.int32)

  def layer(carry, wb):
    g, _, _, _ = carry
    W, b, isfirst = wb
    S = scat(g.reshape(_NQ * N, _DH), src, dst, zeros_acc)

    def f_mid(_):
      h, gn, cs, cq = mid(S, S, g, dis, W, b)
      return h, gn, cs, cq

    def f_last(_):
      h, cs, cq = lastk(S, S, g, dis, b)
      return h, h, cs, cq

    h, gn, cs, cq = lax.cond(isfirst > 0, f_mid, f_last, 0)
    return (gn, h, cs, cq), None

  h0 = jnp.zeros((N, D), jnp.float32)
  c0 = jnp.zeros((1, D), jnp.float32)
  (_, h2, csum, csq), _ = lax.scan(layer, (g1, h0, c0, c0),
                                   (Ws, bs, firsts))

  out = pl.pallas_call(
      _make_tc_final(N),
      grid=(nb,),
      in_specs=[
          pl.BlockSpec((RB, D), row),
          pl.BlockSpec((1, D), full),
          pl.BlockSpec((1, D), full),
      ],
      out_specs=pl.BlockSpec((RB, D), row),
      out_shape=jax.ShapeDtypeStruct((N, D), jnp.float32),
  )(h2, csum, csq)

  return out
